# baseline (speedup 1.0000x reference)
import functools

import jax
import jax.numpy as jnp
from jax import lax
from jax.experimental import pallas as pl
from jax.experimental.pallas import tpu as pltpu
from jax.experimental.pallas import tpu_sc as plsc

_NCLS = 1000
_NB = 1024
_ROWS_PER_BLK = 512
_NW = 16
_L = 16


def _row_stats(yh_ref, y_ref, loss_ref, cls_ref):
    yh = yh_ref[...]
    yv = y_ref[...]
    sumexp = jnp.sum(jnp.exp(yh), axis=1, keepdims=True)
    s_yy = jnp.sum(yv * yh, axis=1, keepdims=True)
    s_y = jnp.sum(yv, axis=1, keepdims=True)
    loss_ref[...] = s_y * jnp.log(sumexp) - s_yy
    mx = jnp.max(yv, axis=1, keepdims=True)
    ids = lax.broadcasted_iota(jnp.int32, yv.shape, 1)
    cls_ref[...] = jnp.min(
        jnp.where(yv == mx, ids, jnp.int32(2**30)), axis=1, keepdims=True)


def _make_segment_mean(batch):
    bpw = _NB // _NW
    unroll = 4
    mesh = plsc.VectorSubcoreMesh(
        core_axis_name="c", subcore_axis_name="s", num_cores=1)

    @functools.partial(
        pl.kernel,
        mesh=mesh,
        compiler_params=pltpu.CompilerParams(needs_layout_passes=False),
        out_type=jax.ShapeDtypeStruct((_NB,), jnp.float32),
        scratch_types=[
            pltpu.VMEM((batch,), jnp.float32),
            pltpu.VMEM((batch,), jnp.int32),
            pltpu.VMEM((bpw,), jnp.float32),
            pltpu.VMEM((bpw,), jnp.float32),
            pltpu.VMEM((bpw,), jnp.float32),
        ],
    )
    def seg(loss_hbm, cls_hbm, out_hbm, loss_v, cls_v, sums_v, cnts_v, res_v):
        sid = lax.axis_index("s")
        lo = sid * bpw
        pltpu.sync_copy(loss_hbm, loss_v)
        pltpu.sync_copy(cls_hbm, cls_v)
        zero = jnp.zeros((_L,), jnp.float32)
        ones = jnp.full((_L,), 1.0, jnp.float32)
        for q in range(bpw // _L):
            sums_v[pl.ds(q * _L, _L)] = zero
            cnts_v[pl.ds(q * _L, _L)] = zero

        def body(j, carry):
            base = j * (_L * unroll)
            for k in range(unroll):
                cv = cls_v[pl.ds(base + k * _L, _L)]
                lv = loss_v[pl.ds(base + k * _L, _L)]
                rel = cv - lo
                mask = (rel >= 0) & (rel < bpw)
                idx = jnp.clip(rel, 0, bpw - 1)
                plsc.addupdate_scatter(sums_v, [idx], lv, mask=mask)
                plsc.addupdate_scatter(cnts_v, [idx], ones, mask=mask)
            return carry

        lax.fori_loop(0, batch // (_L * unroll), body, 0)
        for q in range(bpw // _L):
            sl = pl.ds(q * _L, _L)
            res_v[sl] = sums_v[sl] / cnts_v[sl]
        pltpu.sync_copy(res_v, out_hbm.at[pl.ds(lo, bpw)])

    return seg


def kernel(y_hat, y):
    b, c = y_hat.shape
    r = _ROWS_PER_BLK
    nblk = b // r
    loss, cls = pl.pallas_call(
        _row_stats,
        grid=(nblk,),
        in_specs=[
            pl.BlockSpec((r, c), lambda i: (i, 0)),
            pl.BlockSpec((r, c), lambda i: (i, 0)),
        ],
        out_specs=[
            pl.BlockSpec((r, 1), lambda i: (i, 0)),
            pl.BlockSpec((r, 1), lambda i: (i, 0)),
        ],
        out_shape=[
            jax.ShapeDtypeStruct((b, 1), jnp.float32),
            jax.ShapeDtypeStruct((b, 1), jnp.int32),
        ],
    )(y_hat, y)
    out = _make_segment_mean(b)(loss.reshape(b), cls.reshape(b))
    return out[:c]


# speedup vs baseline: 1.0300x; 1.0300x over previous
import functools

import jax
import jax.numpy as jnp
from jax import lax
from jax.experimental import pallas as pl
from jax.experimental.pallas import tpu as pltpu
from jax.experimental.pallas import tpu_sc as plsc

_NCLS = 1000
_NB = 1024
_ROWS_PER_BLK = 512
_NW = 16
_L = 16


def _row_stats(yh_ref, y_ref, loss_ref, cls_ref):
    yh = yh_ref[...]
    yv = y_ref[...]
    m = jnp.max(yh, axis=1, keepdims=True)
    e = jnp.exp(yh - m)
    lse = jnp.log(jnp.sum(e, axis=1, keepdims=True)) + m
    s_yy = jnp.sum(yv * yh, axis=1, keepdims=True)
    s_y = jnp.sum(yv, axis=1, keepdims=True)
    loss_ref[...] = (s_y * lse - s_yy)[:, 0]
    mx = jnp.max(yv, axis=1, keepdims=True)
    ids = lax.broadcasted_iota(jnp.int32, yv.shape, 1)
    cls_ref[...] = jnp.min(jnp.where(yv == mx, ids, jnp.int32(2**30)), axis=1)


def _make_segment_mean(batch):
    bpw = _NB // _NW
    unroll = 4
    mesh = plsc.VectorSubcoreMesh(
        core_axis_name="c", subcore_axis_name="s", num_cores=1)

    @functools.partial(
        pl.kernel,
        mesh=mesh,
        compiler_params=pltpu.CompilerParams(needs_layout_passes=False),
        out_type=jax.ShapeDtypeStruct((_NB,), jnp.float32),
        scratch_types=[
            pltpu.VMEM((batch,), jnp.float32),
            pltpu.VMEM((batch,), jnp.int32),
            pltpu.VMEM((bpw,), jnp.float32),
            pltpu.VMEM((bpw,), jnp.float32),
            pltpu.VMEM((bpw,), jnp.float32),
        ],
    )
    def seg(loss_hbm, cls_hbm, out_hbm, loss_v, cls_v, sums_v, cnts_v, res_v):
        sid = lax.axis_index("s")
        lo = sid * bpw
        pltpu.sync_copy(loss_hbm, loss_v)
        pltpu.sync_copy(cls_hbm, cls_v)
        zero = jnp.zeros((_L,), jnp.float32)
        ones = jnp.full((_L,), 1.0, jnp.float32)
        for q in range(bpw // _L):
            sums_v[pl.ds(q * _L, _L)] = zero
            cnts_v[pl.ds(q * _L, _L)] = zero

        def body(j, carry):
            base = j * (_L * unroll)
            for k in range(unroll):
                cv = cls_v[pl.ds(base + k * _L, _L)]
                lv = loss_v[pl.ds(base + k * _L, _L)]
                rel = cv - lo
                mask = (rel >= 0) & (rel < bpw)
                idx = jnp.clip(rel, 0, bpw - 1)
                plsc.addupdate_scatter(sums_v, [idx], lv, mask=mask)
                plsc.addupdate_scatter(cnts_v, [idx], ones, mask=mask)
            return carry

        lax.fori_loop(0, batch // (_L * unroll), body, 0)
        for q in range(bpw // _L):
            sl = pl.ds(q * _L, _L)
            res_v[sl] = sums_v[sl] / cnts_v[sl]
        pltpu.sync_copy(res_v, out_hbm.at[pl.ds(lo, bpw)])

    return seg


def kernel(y_hat, y):
    b, c = y_hat.shape
    r = _ROWS_PER_BLK
    nblk = b // r
    loss, cls = pl.pallas_call(
        _row_stats,
        grid=(nblk,),
        in_specs=[
            pl.BlockSpec((r, c), lambda i: (i, 0)),
            pl.BlockSpec((r, c), lambda i: (i, 0)),
        ],
        out_specs=[
            pl.BlockSpec((r,), lambda i: (i,)),
            pl.BlockSpec((r,), lambda i: (i,)),
        ],
        out_shape=[
            jax.ShapeDtypeStruct((b,), jnp.float32),
            jax.ShapeDtypeStruct((b,), jnp.int32),
        ],
    )(y_hat, y)
    out = _make_segment_mean(b)(loss, cls)
    return out[:c]


# speedup vs baseline: 1.0845x; 1.0529x over previous
import functools

import jax
import jax.numpy as jnp
from jax import lax
from jax.experimental import pallas as pl
from jax.experimental.pallas import tpu as pltpu
from jax.experimental.pallas import tpu_sc as plsc

_NCLS = 1000
_NB = 1024
_ROWS_PER_BLK = 1024
_NW = 16
_L = 16


def _row_stats(yh_ref, y_ref, loss_ref, cls_ref):
    yh = yh_ref[...]
    yv = y_ref[...]
    m = jnp.max(yh, axis=1, keepdims=True)
    e = jnp.exp(yh - m)
    lse = jnp.log(jnp.sum(e, axis=1, keepdims=True)) + m
    s_yy = jnp.sum(yv * yh, axis=1, keepdims=True)
    s_y = jnp.sum(yv, axis=1, keepdims=True)
    loss_ref[...] = (s_y * lse - s_yy)[:, 0]
    mx = jnp.max(yv, axis=1, keepdims=True)
    ids = lax.broadcasted_iota(jnp.int32, yv.shape, 1)
    cls_ref[...] = jnp.min(jnp.where(yv == mx, ids, jnp.int32(2**30)), axis=1)


def _make_segment_mean(batch):
    bpw = _NB // _NW
    unroll = 4
    mesh = plsc.VectorSubcoreMesh(
        core_axis_name="c", subcore_axis_name="s", num_cores=1)

    @functools.partial(
        pl.kernel,
        mesh=mesh,
        compiler_params=pltpu.CompilerParams(needs_layout_passes=False),
        out_type=jax.ShapeDtypeStruct((_NB,), jnp.float32),
        scratch_types=[
            pltpu.VMEM((batch,), jnp.float32),
            pltpu.VMEM((batch,), jnp.int32),
            pltpu.VMEM((bpw,), jnp.float32),
            pltpu.VMEM((bpw,), jnp.float32),
            pltpu.VMEM((bpw,), jnp.float32),
        ],
    )
    def seg(loss_hbm, cls_hbm, out_hbm, loss_v, cls_v, sums_v, cnts_v, res_v):
        sid = lax.axis_index("s")
        lo = sid * bpw
        pltpu.sync_copy(loss_hbm, loss_v)
        pltpu.sync_copy(cls_hbm, cls_v)
        zero = jnp.zeros((_L,), jnp.float32)
        ones = jnp.full((_L,), 1.0, jnp.float32)
        for q in range(bpw // _L):
            sums_v[pl.ds(q * _L, _L)] = zero
            cnts_v[pl.ds(q * _L, _L)] = zero

        def body(j, carry):
            base = j * (_L * unroll)
            for k in range(unroll):
                cv = cls_v[pl.ds(base + k * _L, _L)]
                lv = loss_v[pl.ds(base + k * _L, _L)]
                rel = cv - lo
                mask = (rel >= 0) & (rel < bpw)
                idx = jnp.clip(rel, 0, bpw - 1)
                plsc.addupdate_scatter(sums_v, [idx], lv, mask=mask)
                plsc.addupdate_scatter(cnts_v, [idx], ones, mask=mask)
            return carry

        lax.fori_loop(0, batch // (_L * unroll), body, 0)
        for q in range(bpw // _L):
            sl = pl.ds(q * _L, _L)
            res_v[sl] = sums_v[sl] / cnts_v[sl]
        pltpu.sync_copy(res_v, out_hbm.at[pl.ds(lo, bpw)])

    return seg


def kernel(y_hat, y):
    b, c = y_hat.shape
    r = _ROWS_PER_BLK
    nblk = b // r
    loss, cls = pl.pallas_call(
        _row_stats,
        grid=(nblk,),
        in_specs=[
            pl.BlockSpec((r, c), lambda i: (i, 0)),
            pl.BlockSpec((r, c), lambda i: (i, 0)),
        ],
        out_specs=[
            pl.BlockSpec((r,), lambda i: (i,)),
            pl.BlockSpec((r,), lambda i: (i,)),
        ],
        out_shape=[
            jax.ShapeDtypeStruct((b,), jnp.float32),
            jax.ShapeDtypeStruct((b,), jnp.int32),
        ],
    )(y_hat, y)
    out = _make_segment_mean(b)(loss, cls)
    return out[:c]
